# baseline (speedup 1.0000x reference)
import functools

import jax
import jax.numpy as jnp
from jax import lax
from jax.experimental import pallas as pl
from jax.experimental.pallas import tpu as pltpu
from jax.experimental.pallas import tpu_sc as plsc

_K = 8448
_Z = 384
_NVN = 26112
_NCN = 17664
_DEG = 7
_LLR_MAX = 20.0
_NITER = 20
_B = 16

_HALF = _NCN // 2
_CC = 384
_NCHUNK = _HALF // _CC
_GPC = _CC // 16
_BLK = _DEG * _CC
_MSGLEN = _DEG * _HALF
_MC = 6528
_LN2 = 0.6931471805599453
_SIGN = -2147483648


_LOG_C = (-7.989150924009314e-07, 1.0000083697347821, -0.49982348946501404,
          0.3325308523560284, -0.2552298371597656, 0.22039067151343966,
          -0.13766448897562308)


def _log_f32(r):
    bits = lax.bitcast_convert_type(r, jnp.int32)
    e = ((bits >> 23) & 0xFF) - 127
    m = lax.bitcast_convert_type((bits & 0x7FFFFF) | 0x3F800000, jnp.float32)
    big = m >= 1.4142135
    m = jnp.where(big, m * 0.5, m)
    e = jnp.where(big, e + 1, e)
    y = m - 1.0
    p = jnp.float32(_LOG_C[6])
    for coef in _LOG_C[5::-1]:
        p = p * y + jnp.float32(coef)
    return e.astype(jnp.float32) * _LN2 + p


def _phi(x):
    x = jnp.clip(x, 8.5e-8, 16.635532)
    v = jnp.exp(x)
    return _log_f32((v + 1.0) / (v - 1.0))


def _bp_body(llr_hbm, vnt_hbm, out_hbm, acc_hbm, msg_hbm,
             tot_v, acc_v, llr_v, mrg_v,
             idx_a, idx_b, mi_a, mi_b, mo_a, mo_b,
             in_sa, in_sb, out_sa, out_sb):
    c = lax.axis_index("c")
    s = lax.axis_index("s")
    batch = c * 8 + (s // 2)
    half = s % 2
    w = c * 16 + s

    zero16 = jnp.zeros((16,), jnp.float32)

    pltpu.sync_copy(llr_hbm.at[pl.ds(batch * _NVN, _NVN)], llr_v)
    pltpu.sync_copy(llr_hbm.at[pl.ds(batch * _NVN, _NVN)], tot_v)

    def _init(i, carry):
        acc_v[pl.ds(i * 16, 16)] = zero16
        return carry
    lax.fori_loop(0, _NVN // 16, _init, 0)

    def _zbuf(i, carry):
        mo_a[pl.ds(i * 16, 16)] = zero16
        return carry
    lax.fori_loop(0, _BLK // 16, _zbuf, 0)

    def _zchunk(ch, carry):
        pltpu.sync_copy(mo_a, msg_hbm.at[pl.ds(w * _MSGLEN + ch * _BLK, _BLK)])
        return carry
    lax.fori_loop(0, _NCHUNK, _zchunk, 0)

    def _start_in(ch, ib, mb, sem):
        blk = (half * _NCHUNK + ch) * _BLK
        pltpu.make_async_copy(vnt_hbm.at[pl.ds(blk, _BLK)], ib, sem).start()
        pltpu.make_async_copy(
            msg_hbm.at[pl.ds(w * _MSGLEN + ch * _BLK, _BLK)], mb, sem).start()

    def _wait_in(ch, ib, mb, sem):
        blk = (half * _NCHUNK + ch) * _BLK
        pltpu.make_async_copy(vnt_hbm.at[pl.ds(blk, _BLK)], ib, sem).wait()
        pltpu.make_async_copy(
            msg_hbm.at[pl.ds(w * _MSGLEN + ch * _BLK, _BLK)], mb, sem).wait()

    def _start_out(ch, mob, sem):
        pltpu.make_async_copy(
            mob, msg_hbm.at[pl.ds(w * _MSGLEN + ch * _BLK, _BLK)], sem).start()

    def _wait_out(ch, mob, sem):
        pltpu.make_async_copy(
            mob, msg_hbm.at[pl.ds(w * _MSGLEN + ch * _BLK, _BLK)], sem).wait()

    def _iter(it, carry):
        def _compute(idx_v, msg_v, out_v):

            @plsc.parallel_loop(0, _GPC, unroll=2)
            def _grp(g):
                o = g * 16
                idxs = []
                bsg = []
                nus = []
                dus = []
                for j in range(_DEG):
                    idx = idx_v[pl.ds(j * _CC + o, 16)]
                    t = plsc.load_gather(tot_v, [idx])
                    m = t - msg_v[pl.ds(j * _CC + o, 16)]
                    idxs.append(idx)
                    bsg.append(lax.bitcast_convert_type(m, jnp.int32) & _SIGN)
                    x = jnp.minimum(jnp.abs(m), 16.635532)
                    u = jnp.exp(-x)
                    nus.append(1.0 - u)
                    dus.append(1.0 + u)
                pre_n = [nus[0]]
                pre_d = [dus[0]]
                for j in range(1, _DEG - 1):
                    pre_n.append(pre_n[-1] * nus[j])
                    pre_d.append(pre_d[-1] * dus[j])
                suf_n = [None] * _DEG
                suf_d = [None] * _DEG
                suf_n[_DEG - 1] = nus[_DEG - 1]
                suf_d[_DEG - 1] = dus[_DEG - 1]
                for j in range(_DEG - 2, 0, -1):
                    suf_n[j] = suf_n[j + 1] * nus[j]
                    suf_d[j] = suf_d[j + 1] * dus[j]
                b_tot = bsg[0]
                for j in range(1, _DEG):
                    b_tot = b_tot ^ bsg[j]
                for j in range(_DEG):
                    if j == 0:
                        a_j, b_j = suf_n[1], suf_d[1]
                    elif j == _DEG - 1:
                        a_j, b_j = pre_n[_DEG - 2], pre_d[_DEG - 2]
                    else:
                        a_j = pre_n[j - 1] * suf_n[j + 1]
                        b_j = pre_d[j - 1] * suf_d[j + 1]
                    ext = _log_f32((b_j + a_j) / (b_j - a_j))
                    newm = lax.bitcast_convert_type(
                        lax.bitcast_convert_type(ext, jnp.int32)
                        | (b_tot ^ bsg[j]), jnp.float32)
                    out_v[pl.ds(j * _CC + o, 16)] = newm
                    plsc.addupdate_scatter(acc_v, [idxs[j]], newm)

        _start_in(0, idx_a, mi_a, in_sa)
        _start_in(1, idx_b, mi_b, in_sb)

        _wait_in(0, idx_a, mi_a, in_sa)
        _compute(idx_a, mi_a, mo_a)
        _start_out(0, mo_a, out_sa)
        _start_in(2, idx_a, mi_a, in_sa)

        _wait_in(1, idx_b, mi_b, in_sb)
        _compute(idx_b, mi_b, mo_b)
        _start_out(1, mo_b, out_sb)
        _start_in(3, idx_b, mi_b, in_sb)

        def _pair(k, carry2):
            ch0 = 2 * k
            _wait_in(ch0, idx_a, mi_a, in_sa)
            _wait_out(ch0 - 2, mo_a, out_sa)
            _compute(idx_a, mi_a, mo_a)
            _start_out(ch0, mo_a, out_sa)
            _start_in(ch0 + 2, idx_a, mi_a, in_sa)
            ch1 = 2 * k + 1
            _wait_in(ch1, idx_b, mi_b, in_sb)
            _wait_out(ch1 - 2, mo_b, out_sb)
            _compute(idx_b, mi_b, mo_b)
            _start_out(ch1, mo_b, out_sb)
            _start_in(ch1 + 2, idx_b, mi_b, in_sb)
            return carry2
        lax.fori_loop(1, 10, _pair, 0)

        _wait_in(20, idx_a, mi_a, in_sa)
        _wait_out(18, mo_a, out_sa)
        _compute(idx_a, mi_a, mo_a)
        _start_out(20, mo_a, out_sa)
        _start_in(22, idx_a, mi_a, in_sa)

        _wait_in(21, idx_b, mi_b, in_sb)
        _wait_out(19, mo_b, out_sb)
        _compute(idx_b, mi_b, mo_b)
        _start_out(21, mo_b, out_sb)

        _wait_in(22, idx_a, mi_a, in_sa)
        _wait_out(20, mo_a, out_sa)
        _compute(idx_a, mi_a, mo_a)
        _start_out(22, mo_a, out_sa)

        _wait_out(21, mo_b, out_sb)
        _wait_out(22, mo_a, out_sa)

        pltpu.sync_copy(acc_v, acc_hbm.at[pl.ds(w * _NVN, _NVN)])
        plsc.subcore_barrier()

        def _mchunk(mc, carry2):
            pltpu.sync_copy(
                acc_hbm.at[pl.ds((w ^ 1) * _NVN + mc * _MC, _MC)], mrg_v)

            @plsc.parallel_loop(0, _MC // 16, unroll=4)
            def _madd(i):
                sl = pl.ds(mc * _MC + i * 16, 16)
                tot_v[sl] = llr_v[sl] + acc_v[sl] + mrg_v[pl.ds(i * 16, 16)]
                acc_v[sl] = zero16
            return carry2
        lax.fori_loop(0, _NVN // _MC, _mchunk, 0)
        plsc.subcore_barrier()
        return carry
    lax.fori_loop(0, _NITER, _iter, 0)

    @pl.when(half == 0)
    def _emit():
        def _neg(i, carry):
            sl = pl.ds(i * 16, 16)
            acc_v[sl] = -tot_v[sl]
            return carry
        lax.fori_loop(0, _K // 16, _neg, 0)
        pltpu.sync_copy(acc_v.at[pl.ds(0, _K)],
                        out_hbm.at[pl.ds(batch * _K, _K)])


@functools.lru_cache(maxsize=1)
def _build_bp_kernel():
    @functools.partial(
        pl.kernel,
        out_type=(jax.ShapeDtypeStruct((_B * _K,), jnp.float32),
                  jax.ShapeDtypeStruct((32 * _NVN,), jnp.float32),
                  jax.ShapeDtypeStruct((32 * _MSGLEN,), jnp.float32)),
        mesh=plsc.VectorSubcoreMesh(core_axis_name="c", subcore_axis_name="s"),
        compiler_params=pltpu.CompilerParams(needs_layout_passes=False),
        scratch_types=[
            pltpu.VMEM((_NVN,), jnp.float32),
            pltpu.VMEM((_NVN,), jnp.float32),
            pltpu.VMEM((_NVN,), jnp.float32),
            pltpu.VMEM((_MC,), jnp.float32),
            pltpu.VMEM((_BLK,), jnp.int32),
            pltpu.VMEM((_BLK,), jnp.int32),
            pltpu.VMEM((_BLK,), jnp.float32),
            pltpu.VMEM((_BLK,), jnp.float32),
            pltpu.VMEM((_BLK,), jnp.float32),
            pltpu.VMEM((_BLK,), jnp.float32),
            pltpu.SemaphoreType.DMA,
            pltpu.SemaphoreType.DMA,
            pltpu.SemaphoreType.DMA,
            pltpu.SemaphoreType.DMA,
        ],
    )
    def _bp_kernel(llr_hbm, vnt_hbm, out_hbm, acc_hbm, msg_hbm, *scratch):
        _bp_body(llr_hbm, vnt_hbm, out_hbm, acc_hbm, msg_hbm, *scratch)
    return _bp_kernel


def kernel(llr_ch, cn_idx, vn_idx):
    del cn_idx
    b = llr_ch.shape[0]
    llr_5g = jnp.concatenate(
        [jnp.zeros((b, 2 * _Z), llr_ch.dtype), llr_ch], axis=-1)
    llr_int = (-jnp.clip(llr_5g, -_LLR_MAX, _LLR_MAX)).astype(jnp.float32)
    vnt = vn_idx.astype(jnp.int32).reshape(_NCN, _DEG).T
    vnt_blk = vnt.reshape(_DEG, 2, _NCHUNK, _CC).transpose(1, 2, 0, 3)
    out, _, _ = _build_bp_kernel()(llr_int.reshape(-1), vnt_blk.reshape(-1))
    return out.reshape(_B, _K)


# speedup vs baseline: 1.2380x; 1.2380x over previous
import functools

import jax
import jax.numpy as jnp
from jax import lax
from jax.experimental import pallas as pl
from jax.experimental.pallas import tpu as pltpu
from jax.experimental.pallas import tpu_sc as plsc

_K = 8448
_Z = 384
_NVN = 26112
_NCN = 17664
_DEG = 7
_LLR_MAX = 20.0
_NITER = 20
_B = 16

_HALF = _NCN // 2
_CC = 384
_NCHUNK = _HALF // _CC
_GPC = _CC // 16
_BLK = _DEG * _CC
_MSGLEN = _DEG * _HALF
_MC = 6528
_LN2 = 0.6931471805599453
_SIGN = -2147483648


_LOG_C = (-7.989150924009314e-07, 1.0000083697347821, -0.49982348946501404,
          0.3325308523560284, -0.2552298371597656, 0.22039067151343966,
          -0.13766448897562308)


def _log_f32(r):
    bits = lax.bitcast_convert_type(r, jnp.int32)
    e = ((bits >> 23) & 0xFF) - 127
    m = lax.bitcast_convert_type((bits & 0x7FFFFF) | 0x3F800000, jnp.float32)
    big = m >= 1.4142135
    m = jnp.where(big, m * 0.5, m)
    e = jnp.where(big, e + 1, e)
    y = m - 1.0
    p = jnp.float32(_LOG_C[6])
    for coef in _LOG_C[5::-1]:
        p = p * y + jnp.float32(coef)
    return e.astype(jnp.float32) * _LN2 + p


def _phi(x):
    x = jnp.clip(x, 8.5e-8, 16.635532)
    v = jnp.exp(x)
    return _log_f32((v + 1.0) / (v - 1.0))


def _bp_body(llr_hbm, vnt_hbm, out_hbm, acc_hbm, msg_hbm,
             tot_v, acc_v, llr_v, mrg_v,
             idx_a, idx_b, mi_a, mi_b, mo_a, mo_b,
             in_sa, in_sb, out_sa, out_sb):
    c = lax.axis_index("c")
    s = lax.axis_index("s")
    batch = c * 8 + (s // 2)
    half = s % 2
    w = c * 16 + s

    zero16 = jnp.zeros((16,), jnp.float32)

    pltpu.sync_copy(llr_hbm.at[pl.ds(batch * _NVN, _NVN)], llr_v)
    pltpu.sync_copy(llr_hbm.at[pl.ds(batch * _NVN, _NVN)], tot_v)

    def _init(i, carry):
        acc_v[pl.ds(i * 16, 16)] = zero16
        return carry
    lax.fori_loop(0, _NVN // 16, _init, 0)

    def _zbuf(i, carry):
        mo_a[pl.ds(i * 16, 16)] = zero16
        return carry
    lax.fori_loop(0, _BLK // 16, _zbuf, 0)

    def _zchunk(ch, carry):
        pltpu.sync_copy(mo_a, msg_hbm.at[pl.ds(w * _MSGLEN + ch * _BLK, _BLK)])
        return carry
    lax.fori_loop(0, _NCHUNK, _zchunk, 0)

    def _start_in(ch, ib, mb, sem):
        blk = (half * _NCHUNK + ch) * _BLK
        pltpu.make_async_copy(vnt_hbm.at[pl.ds(blk, _BLK)], ib, sem).start()
        pltpu.make_async_copy(
            msg_hbm.at[pl.ds(w * _MSGLEN + ch * _BLK, _BLK)], mb, sem).start()

    def _wait_in(ch, ib, mb, sem):
        blk = (half * _NCHUNK + ch) * _BLK
        pltpu.make_async_copy(vnt_hbm.at[pl.ds(blk, _BLK)], ib, sem).wait()
        pltpu.make_async_copy(
            msg_hbm.at[pl.ds(w * _MSGLEN + ch * _BLK, _BLK)], mb, sem).wait()

    def _start_out(ch, mob, sem):
        pltpu.make_async_copy(
            mob, msg_hbm.at[pl.ds(w * _MSGLEN + ch * _BLK, _BLK)], sem).start()

    def _wait_out(ch, mob, sem):
        pltpu.make_async_copy(
            mob, msg_hbm.at[pl.ds(w * _MSGLEN + ch * _BLK, _BLK)], sem).wait()

    def _iter(it, carry):
        def _compute(idx_v, msg_v, out_v):

            @plsc.parallel_loop(0, _GPC, unroll=2)
            def _grp(g):
                o = g * 16
                idxs = []
                bsg = []
                nus = []
                dus = []
                for j in range(_DEG):
                    idx = idx_v[pl.ds(j * _CC + o, 16)]
                    t = plsc.load_gather(tot_v, [idx])
                    m = t - msg_v[pl.ds(j * _CC + o, 16)]
                    idxs.append(idx)
                    bsg.append(lax.bitcast_convert_type(m, jnp.int32) & _SIGN)
                    x = jnp.clip(jnp.abs(m), 8.5e-8, 16.635532)
                    u = jnp.exp(-x)
                    nus.append(1.0 - u)
                    dus.append(1.0 + u)
                pre_n = [nus[0]]
                pre_d = [dus[0]]
                for j in range(1, _DEG - 1):
                    pre_n.append(pre_n[-1] * nus[j])
                    pre_d.append(pre_d[-1] * dus[j])
                suf_n = [None] * _DEG
                suf_d = [None] * _DEG
                suf_n[_DEG - 1] = nus[_DEG - 1]
                suf_d[_DEG - 1] = dus[_DEG - 1]
                for j in range(_DEG - 2, 0, -1):
                    suf_n[j] = suf_n[j + 1] * nus[j]
                    suf_d[j] = suf_d[j + 1] * dus[j]
                b_tot = bsg[0]
                for j in range(1, _DEG):
                    b_tot = b_tot ^ bsg[j]
                for j in range(_DEG):
                    if j == 0:
                        a_j, b_j = suf_n[1], suf_d[1]
                    elif j == _DEG - 1:
                        a_j, b_j = pre_n[_DEG - 2], pre_d[_DEG - 2]
                    else:
                        a_j = pre_n[j - 1] * suf_n[j + 1]
                        b_j = pre_d[j - 1] * suf_d[j + 1]
                    ext = _log_f32((b_j + a_j) / (b_j - a_j))
                    newm = lax.bitcast_convert_type(
                        lax.bitcast_convert_type(ext, jnp.int32)
                        | (b_tot ^ bsg[j]), jnp.float32)
                    out_v[pl.ds(j * _CC + o, 16)] = newm
                    plsc.addupdate_scatter(acc_v, [idxs[j]], newm)

        _start_in(0, idx_a, mi_a, in_sa)
        _start_in(1, idx_b, mi_b, in_sb)

        _wait_in(0, idx_a, mi_a, in_sa)
        _compute(idx_a, mi_a, mo_a)
        _start_out(0, mo_a, out_sa)
        _start_in(2, idx_a, mi_a, in_sa)

        _wait_in(1, idx_b, mi_b, in_sb)
        _compute(idx_b, mi_b, mo_b)
        _start_out(1, mo_b, out_sb)
        _start_in(3, idx_b, mi_b, in_sb)

        def _pair(k, carry2):
            ch0 = 2 * k
            _wait_in(ch0, idx_a, mi_a, in_sa)
            _wait_out(ch0 - 2, mo_a, out_sa)
            _compute(idx_a, mi_a, mo_a)
            _start_out(ch0, mo_a, out_sa)
            _start_in(ch0 + 2, idx_a, mi_a, in_sa)
            ch1 = 2 * k + 1
            _wait_in(ch1, idx_b, mi_b, in_sb)
            _wait_out(ch1 - 2, mo_b, out_sb)
            _compute(idx_b, mi_b, mo_b)
            _start_out(ch1, mo_b, out_sb)
            _start_in(ch1 + 2, idx_b, mi_b, in_sb)
            return carry2
        lax.fori_loop(1, 10, _pair, 0)

        _wait_in(20, idx_a, mi_a, in_sa)
        _wait_out(18, mo_a, out_sa)
        _compute(idx_a, mi_a, mo_a)
        _start_out(20, mo_a, out_sa)
        _start_in(22, idx_a, mi_a, in_sa)

        _wait_in(21, idx_b, mi_b, in_sb)
        _wait_out(19, mo_b, out_sb)
        _compute(idx_b, mi_b, mo_b)
        _start_out(21, mo_b, out_sb)

        _wait_in(22, idx_a, mi_a, in_sa)
        _wait_out(20, mo_a, out_sa)
        _compute(idx_a, mi_a, mo_a)
        _start_out(22, mo_a, out_sa)

        _wait_out(21, mo_b, out_sb)
        _wait_out(22, mo_a, out_sa)

        pltpu.sync_copy(acc_v, acc_hbm.at[pl.ds(w * _NVN, _NVN)])
        plsc.subcore_barrier()

        def _mchunk(mc, carry2):
            pltpu.sync_copy(
                acc_hbm.at[pl.ds((w ^ 1) * _NVN + mc * _MC, _MC)], mrg_v)

            @plsc.parallel_loop(0, _MC // 16, unroll=4)
            def _madd(i):
                sl = pl.ds(mc * _MC + i * 16, 16)
                tot_v[sl] = llr_v[sl] + acc_v[sl] + mrg_v[pl.ds(i * 16, 16)]
                acc_v[sl] = zero16
            return carry2
        lax.fori_loop(0, _NVN // _MC, _mchunk, 0)
        plsc.subcore_barrier()
        return carry
    lax.fori_loop(0, _NITER, _iter, 0)

    @pl.when(half == 0)
    def _emit():
        def _neg(i, carry):
            sl = pl.ds(i * 16, 16)
            acc_v[sl] = -tot_v[sl]
            return carry
        lax.fori_loop(0, _K // 16, _neg, 0)
        pltpu.sync_copy(acc_v.at[pl.ds(0, _K)],
                        out_hbm.at[pl.ds(batch * _K, _K)])


@functools.lru_cache(maxsize=1)
def _build_bp_kernel():
    @functools.partial(
        pl.kernel,
        out_type=(jax.ShapeDtypeStruct((_B * _K,), jnp.float32),
                  jax.ShapeDtypeStruct((32 * _NVN,), jnp.float32),
                  jax.ShapeDtypeStruct((32 * _MSGLEN,), jnp.float32)),
        mesh=plsc.VectorSubcoreMesh(core_axis_name="c", subcore_axis_name="s"),
        compiler_params=pltpu.CompilerParams(needs_layout_passes=False),
        scratch_types=[
            pltpu.VMEM((_NVN,), jnp.float32),
            pltpu.VMEM((_NVN,), jnp.float32),
            pltpu.VMEM((_NVN,), jnp.float32),
            pltpu.VMEM((_MC,), jnp.float32),
            pltpu.VMEM((_BLK,), jnp.int32),
            pltpu.VMEM((_BLK,), jnp.int32),
            pltpu.VMEM((_BLK,), jnp.float32),
            pltpu.VMEM((_BLK,), jnp.float32),
            pltpu.VMEM((_BLK,), jnp.float32),
            pltpu.VMEM((_BLK,), jnp.float32),
            pltpu.SemaphoreType.DMA,
            pltpu.SemaphoreType.DMA,
            pltpu.SemaphoreType.DMA,
            pltpu.SemaphoreType.DMA,
        ],
    )
    def _bp_kernel(llr_hbm, vnt_hbm, out_hbm, acc_hbm, msg_hbm, *scratch):
        _bp_body(llr_hbm, vnt_hbm, out_hbm, acc_hbm, msg_hbm, *scratch)
    return _bp_kernel


def kernel(llr_ch, cn_idx, vn_idx):
    del cn_idx
    b = llr_ch.shape[0]
    llr_5g = jnp.concatenate(
        [jnp.zeros((b, 2 * _Z), llr_ch.dtype), llr_ch], axis=-1)
    llr_int = (-jnp.clip(llr_5g, -_LLR_MAX, _LLR_MAX)).astype(jnp.float32)
    vnt = vn_idx.astype(jnp.int32).reshape(_NCN, _DEG).T
    vnt_blk = vnt.reshape(_DEG, 2, _NCHUNK, _CC).transpose(1, 2, 0, 3)
    out, _, _ = _build_bp_kernel()(llr_int.reshape(-1), vnt_blk.reshape(-1))
    return out.reshape(_B, _K)
